# baseline (speedup 1.0000x reference)
import jax
import jax.numpy as jnp
from jax import lax
from jax.experimental import pallas as pl
from jax.experimental.pallas import tpu as pltpu
from jax.experimental.pallas import tpu_sc as plsc

N_NODES = 50000
H = 32
NW = 32
NPASS = 2
R = 784
HALF = NW * R
N_PAD = NPASS * HALF
RPAD = R + 8
E = 1600000
CH = 4096
NV = CH // 16
NCHUNK = 392
E_PAD = NCHUNK * CH
B = 256
HB = 4352

_mesh = plsc.VectorSubcoreMesh(core_axis_name="c", subcore_axis_name="s")


def _sc_body(feat, srcp, dstp, s_out, s2_out, mx_out, mn_out, cnt_out,
             dst_buf, sem2, pcnt_buf, pbase_buf, hs_buf, hd_buf, hdsh_buf,
             srcv_buf, rows_buf, mx_acc, mn_acc, cnt_acc,
             s_sh, s2_sh, sem):
    c = lax.axis_index("c")
    s = lax.axis_index("s")
    wid = s * 2 + c
    soff = s * RPAD

    lane = lax.iota(jnp.int32, 16)
    lane2 = lane + 16
    lane0 = lane == 0
    fones = jnp.full((16,), 1.0, jnp.float32)
    zf = jnp.zeros((16,), jnp.float32)
    zi = jnp.zeros((16,), jnp.int32)
    NEG = jnp.full((16,), -3.0e38, jnp.float32)
    POS = jnp.full((16,), 3.0e38, jnp.float32)

    _dnums = lax.GatherDimensionNumbers(
        offset_dims=(), collapsed_slice_dims=(0,), start_index_map=(0,))

    def _bcast(v, jj):
        idx = jnp.full((16, 1), jj, jnp.int32)
        return lax.gather(v, idx, _dnums, (1,),
                          mode=lax.GatherScatterMode.PROMISE_IN_BOUNDS)

    def _splat(x):
        return jnp.full((16,), x, jnp.int32)

    def init_hs(j, _):
        hs_buf[pl.ds(j * 16, 16)] = zi
        return 0
    lax.fori_loop(0, HB // 16, init_hs, 0)

    def one_pass(p, _):
        lo = pl.multiple_of(p * HALF + wid * R, 16)
        dump = R + soff

        def init_acc(j, _):
            mx_acc[j, pl.ds(0, 16)] = NEG
            mx_acc[j, pl.ds(16, 16)] = NEG
            mn_acc[j, pl.ds(0, 16)] = POS
            mn_acc[j, pl.ds(16, 16)] = POS
            return 0
        lax.fori_loop(0, R, init_acc, 0)

        def init_cnt(j, _):
            cnt_acc[pl.ds(j * 16, 16)] = zf
            return 0
        lax.fori_loop(0, 50, init_cnt, 0)

        def init_rows(j, _):
            rows_buf[j, pl.ds(0, 16)] = zf
            rows_buf[j, pl.ds(16, 16)] = zf
            return 0
        lax.fori_loop(0, B, init_rows, 0)

        for k in range(3):
            pltpu.sync_copy(rows_buf, s_sh.at[pl.ds(soff + k * B, B)])
            pltpu.sync_copy(rows_buf, s2_sh.at[pl.ds(soff + k * B, B)])
        rem = RPAD - 3 * B
        pltpu.sync_copy(rows_buf.at[pl.ds(0, rem)],
                        s_sh.at[pl.ds(soff + 3 * B, rem)])
        pltpu.sync_copy(rows_buf.at[pl.ds(0, rem)],
                        s2_sh.at[pl.ds(soff + 3 * B, rem)])

        def flush(fofs):
            fofs = pl.multiple_of(fofs, B)
            def mkw(j, _):
                hdsh_buf[pl.ds(j * 16, 16)] = hd_buf[pl.ds(fofs + j * 16, 16)]
                return 0
            lax.fori_loop(0, B // 16, mkw, 0)

            pltpu.async_copy(srcp.at[hs_buf.at[pl.ds(fofs, B)]],
                             srcv_buf, sem).wait()
            pltpu.async_copy(feat.at[srcv_buf], rows_buf, sem).wait()

            sum_cp = pltpu.async_copy(rows_buf, s_sh.at[hdsh_buf], sem,
                                      add=True)

            def edge_vreg(jv, _):
                hdv = hd_buf[pl.ds(fofs + jv * 16, 16)] - soff
                for jj in range(16):
                    d_splat = _bcast(hdv, jj)
                    j = jv * 16 + jj
                    m0 = rows_buf[j, pl.ds(0, 16)]
                    m1 = rows_buf[j, pl.ds(16, 16)]
                    a0 = plsc.load_gather(mx_acc, [d_splat, lane])
                    plsc.store_scatter(mx_acc, [d_splat, lane],
                                       jnp.maximum(a0, m0))
                    a1 = plsc.load_gather(mx_acc, [d_splat, lane2])
                    plsc.store_scatter(mx_acc, [d_splat, lane2],
                                       jnp.maximum(a1, m1))
                    b0 = plsc.load_gather(mn_acc, [d_splat, lane])
                    plsc.store_scatter(mn_acc, [d_splat, lane],
                                       jnp.minimum(b0, m0))
                    b1 = plsc.load_gather(mn_acc, [d_splat, lane2])
                    plsc.store_scatter(mn_acc, [d_splat, lane2],
                                       jnp.minimum(b1, m1))
                    plsc.addupdate_scatter(cnt_acc, [d_splat], fones,
                                           mask=lane0)
                return 0
            lax.fori_loop(0, B // 16, edge_vreg, 0)

            sum_cp.wait()

            @plsc.parallel_loop(0, B, unroll=4)
            def sq_loop(j):
                m0 = rows_buf[j, pl.ds(0, 16)]
                m1 = rows_buf[j, pl.ds(16, 16)]
                rows_buf[j, pl.ds(0, 16)] = m0 * m0
                rows_buf[j, pl.ds(16, 16)] = m1 * m1

            pltpu.sync_copy(rows_buf, s2_sh.at[hdsh_buf], add=True)

        pltpu.async_copy(dstp.at[pl.ds(0, CH)], dst_buf.at[pl.ds(0, CH)],
                         sem2)

        def chunk_body(ci, ptr):
            par = pl.multiple_of((ci & 1) * CH, CH)
            cofs = pl.multiple_of(ci * CH, CH)
            pltpu.make_async_copy(dstp.at[pl.ds(cofs, CH)],
                                  dst_buf.at[pl.ds(par, CH)], sem2).wait()

            @pl.when(ci + 1 < NCHUNK)
            def _prefetch():
                nofs = pl.multiple_of((ci + 1) * CH, CH)
                npar = pl.multiple_of(((ci + 1) & 1) * CH, CH)
                pltpu.async_copy(dstp.at[pl.ds(nofs, CH)],
                                 dst_buf.at[pl.ds(npar, CH)], sem2)

            @plsc.parallel_loop(0, NV // 16, unroll=3)
            def l1(g):
                gofs = pl.multiple_of(par + g * 256, 256)
                acc = zi
                for k in range(16):
                    u = dst_buf[pl.ds(gofs + k * 16, 16)] - lo
                    m = (u >= 0) & (u < R)
                    pop = plsc.all_reduce_population_count(m)
                    acc = jnp.where(lane == k, pop, acc)
                pcnt_buf[pl.ds(g * 16, 16)] = acc

            def l2(g, carry):
                cg = pcnt_buf[pl.ds(g * 16, 16)]
                inc = plsc.cumsum(cg)
                pbase_buf[pl.ds(g * 16, 16)] = inc - cg + carry
                return carry + _bcast(inc, 15)
            tot = lax.fori_loop(0, NV // 16, l2, _splat(ptr))

            @plsc.parallel_loop(0, NV // 16, unroll=3)
            def l3(g):
                gofs = pl.multiple_of(par + g * 256, 256)
                pb = pbase_buf[pl.ds(g * 16, 16)]
                for k in range(16):
                    u = dst_buf[pl.ds(gofs + k * 16, 16)] - lo
                    m = (u >= 0) & (u < R)
                    inc = plsc.cumsum(jnp.where(m, 1, 0))
                    pos = _bcast(pb, k) + inc - 1
                    eid = _splat(cofs + g * 256 + k * 16) + lane
                    plsc.store_scatter(hs_buf, [pos], eid, mask=m)
                    plsc.store_scatter(hd_buf, [pos], u + soff, mask=m)

            ptr_new = jnp.sum(jnp.where(lane == 15, tot, 0))

            def drain_cond(st):
                return st[0] + B <= st[1]

            def drain_body(st):
                fofs, pn = st
                flush(fofs)
                return (fofs + B, pn)
            fofs, _ = lax.while_loop(drain_cond, drain_body,
                                     (jnp.int32(0), ptr_new))

            def compact(q):
                q = pl.multiple_of(q, B)
                def cp(j, _):
                    hs_buf[pl.ds(j * 16, 16)] = hs_buf[pl.ds(q + j * 16, 16)]
                    hd_buf[pl.ds(j * 16, 16)] = hd_buf[pl.ds(q + j * 16, 16)]
                    return 0
                lax.fori_loop(0, B // 16, cp, 0)
                return jnp.int32(0)
            fofs = lax.cond(fofs > 0, compact, lambda q: q, fofs)
            return ptr_new - (ptr_new // B) * B

        ptr = lax.fori_loop(0, NCHUNK, chunk_body, jnp.int32(0))

        def final_flush(q):
            def fix(j, _):
                gidx = lane + j * 16
                hd_buf[pl.ds(j * 16, 16)] = jnp.where(
                    gidx >= q, dump, hd_buf[pl.ds(j * 16, 16)])
                return 0
            lax.fori_loop(0, B // 16, fix, 0)
            flush(jnp.int32(0))
            return jnp.int32(0)
        ptr = lax.cond(ptr > 0, final_flush, lambda q: q, ptr)

        pltpu.sync_copy(mx_acc.at[pl.ds(0, R)], mx_out.at[pl.ds(lo, R)])
        pltpu.sync_copy(mn_acc.at[pl.ds(0, R)], mn_out.at[pl.ds(lo, R)])
        pltpu.sync_copy(cnt_acc.at[pl.ds(0, R)], cnt_out.at[pl.ds(lo, R)])
        pltpu.sync_copy(s_sh.at[pl.ds(soff, R)], s_out.at[pl.ds(lo, R)])
        pltpu.sync_copy(s2_sh.at[pl.ds(soff, R)], s2_out.at[pl.ds(lo, R)])
        return 0

    lax.fori_loop(0, NPASS, one_pass, 0)


_sc_call = pl.kernel(
    _sc_body,
    out_type=[
        jax.ShapeDtypeStruct((N_PAD, H), jnp.float32),
        jax.ShapeDtypeStruct((N_PAD, H), jnp.float32),
        jax.ShapeDtypeStruct((N_PAD, H), jnp.float32),
        jax.ShapeDtypeStruct((N_PAD, H), jnp.float32),
        jax.ShapeDtypeStruct((N_PAD,), jnp.float32),
    ],
    mesh=_mesh,
    compiler_params=pltpu.CompilerParams(
        use_tc_tiling_on_sc=False, needs_layout_passes=False),
    scratch_types=[
        pltpu.VMEM((2 * CH,), jnp.int32),
        pltpu.SemaphoreType.DMA,
        pltpu.VMEM((NV,), jnp.int32),
        pltpu.VMEM((NV,), jnp.int32),
        pltpu.VMEM((HB,), jnp.int32),
        pltpu.VMEM((HB,), jnp.int32),
        pltpu.VMEM((B,), jnp.int32),
        pltpu.VMEM((B,), jnp.int32),
        pltpu.VMEM((B, H), jnp.float32),
        pltpu.VMEM((R + 8, H), jnp.float32),
        pltpu.VMEM((R + 8, H), jnp.float32),
        pltpu.VMEM((800,), jnp.float32),
        pltpu.VMEM_SHARED((16 * RPAD, H), jnp.float32),
        pltpu.VMEM_SHARED((16 * RPAD, H), jnp.float32),
        pltpu.SemaphoreType.DMA,
    ],
)


def _tc_body(s_ref, s2_ref, mx_ref, mn_ref, cnt_ref, w_ref, b_ref, o_ref):
    cnt = cnt_ref[...]
    denom = jnp.maximum(cnt, 1.0)
    has = cnt > 0.0
    mean = s_ref[...] / denom
    mx = jnp.where(has, mx_ref[...], 0.0)
    mn = jnp.where(has, mn_ref[...], 0.0)
    msq = s2_ref[...] / denom
    var = jnp.maximum(msq - mean * mean, 0.0)
    std = jnp.sqrt(var + 1e-5)
    h = jnp.concatenate([mean, mx, mn, std], axis=1)
    o = jnp.dot(h, w_ref[...], preferred_element_type=jnp.float32) + b_ref[...]
    o_ref[...] = jnp.maximum(o, 0.0)


BN = 512
_tc_call = pl.pallas_call(
    _tc_body,
    grid=(N_PAD // BN,),
    in_specs=[
        pl.BlockSpec((BN, H), lambda i: (i, 0)),
        pl.BlockSpec((BN, H), lambda i: (i, 0)),
        pl.BlockSpec((BN, H), lambda i: (i, 0)),
        pl.BlockSpec((BN, H), lambda i: (i, 0)),
        pl.BlockSpec((BN, 1), lambda i: (i, 0)),
        pl.BlockSpec((4 * H, H), lambda i: (0, 0)),
        pl.BlockSpec((1, H), lambda i: (0, 0)),
    ],
    out_specs=pl.BlockSpec((BN, H), lambda i: (i, 0)),
    out_shape=jax.ShapeDtypeStruct((N_PAD, H), jnp.float32),
)


def kernel(feat, edge_index, W, b):
    src = edge_index[0].astype(jnp.int32)
    dst = edge_index[1].astype(jnp.int32)
    pad = E_PAD - E
    srcp = jnp.concatenate([src, jnp.zeros((pad,), jnp.int32)])
    dstp = jnp.concatenate([dst, jnp.full((pad,), N_PAD - 1, jnp.int32)])
    s_, s2_, mx_, mn_, cnt_ = _sc_call(feat, srcp, dstp)
    out = _tc_call(s_, s2_, mx_, mn_, cnt_.reshape(N_PAD, 1), W,
                   b.reshape(1, H))
    return out[:N_NODES]


# speedup vs baseline: 1.3627x; 1.3627x over previous
import jax
import jax.numpy as jnp
from jax import lax
from jax.experimental import pallas as pl
from jax.experimental.pallas import tpu as pltpu
from jax.experimental.pallas import tpu_sc as plsc

N_NODES = 50000
H = 32
NW = 32
NPASS = 2
R = 784
HALF = NW * R
N_PAD = NPASS * HALF
RPAD = R + 8
E = 1600000
CH = 4096
NV = CH // 16
NCHUNK = 392
E_PAD = NCHUNK * CH
B = 256
HB = 4352

_mesh = plsc.VectorSubcoreMesh(core_axis_name="c", subcore_axis_name="s")


def _sc_body(feat, srcp, dstp, s_out, s2_out, mx_out, mn_out, cnt_out,
             dst_buf, sem2, pcnt_buf, pbase_buf, hs_buf, hd_buf, hdsh_buf,
             srcv_buf, rows_buf, mx_acc, mn_acc, cnt_acc,
             s_sh, s2_sh, sem):
    c = lax.axis_index("c")
    s = lax.axis_index("s")
    wid = s * 2 + c
    soff = s * RPAD

    lane = lax.iota(jnp.int32, 16)
    lane2 = lane + 16
    lane0 = lane == 0
    fones = jnp.full((16,), 1.0, jnp.float32)
    zf = jnp.zeros((16,), jnp.float32)
    zi = jnp.zeros((16,), jnp.int32)
    NEG = jnp.full((16,), -3.0e38, jnp.float32)
    POS = jnp.full((16,), 3.0e38, jnp.float32)

    _dnums = lax.GatherDimensionNumbers(
        offset_dims=(), collapsed_slice_dims=(0,), start_index_map=(0,))

    def _bcast(v, jj):
        idx = jnp.full((16, 1), jj, jnp.int32)
        return lax.gather(v, idx, _dnums, (1,),
                          mode=lax.GatherScatterMode.PROMISE_IN_BOUNDS)

    def _splat(x):
        return jnp.full((16,), x, jnp.int32)

    def init_hs(j, _):
        hs_buf[pl.ds(j * 16, 16)] = zi
        return 0
    lax.fori_loop(0, HB // 16, init_hs, 0)

    def one_pass(p, _):
        lo = pl.multiple_of(p * HALF + wid * R, 16)
        dump = R + soff

        def init_acc(j, _):
            mx_acc[j, pl.ds(0, 16)] = NEG
            mx_acc[j, pl.ds(16, 16)] = NEG
            mn_acc[j, pl.ds(0, 16)] = POS
            mn_acc[j, pl.ds(16, 16)] = POS
            return 0
        lax.fori_loop(0, R, init_acc, 0)

        def init_cnt(j, _):
            cnt_acc[pl.ds(j * 16, 16)] = zf
            return 0
        lax.fori_loop(0, 50, init_cnt, 0)

        def init_rows(j, _):
            rows_buf[j, pl.ds(0, 16)] = zf
            rows_buf[j, pl.ds(16, 16)] = zf
            return 0
        lax.fori_loop(0, B, init_rows, 0)

        for k in range(3):
            pltpu.sync_copy(rows_buf, s_sh.at[pl.ds(soff + k * B, B)])
            pltpu.sync_copy(rows_buf, s2_sh.at[pl.ds(soff + k * B, B)])
        rem = RPAD - 3 * B
        pltpu.sync_copy(rows_buf.at[pl.ds(0, rem)],
                        s_sh.at[pl.ds(soff + 3 * B, rem)])
        pltpu.sync_copy(rows_buf.at[pl.ds(0, rem)],
                        s2_sh.at[pl.ds(soff + 3 * B, rem)])

        def flush(fofs):
            fofs = pl.multiple_of(fofs, B)
            def mkw(j, _):
                hdsh_buf[pl.ds(j * 16, 16)] = hd_buf[pl.ds(fofs + j * 16, 16)]
                return 0
            lax.fori_loop(0, B // 16, mkw, 0)

            pltpu.async_copy(srcp.at[hs_buf.at[pl.ds(fofs, B)]],
                             srcv_buf, sem).wait()
            pltpu.async_copy(feat.at[srcv_buf], rows_buf, sem).wait()

            sum_cp = pltpu.async_copy(rows_buf, s_sh.at[hdsh_buf], sem,
                                      add=True)

            def edge_vreg(jv, _):
                hdv = hd_buf[pl.ds(fofs + jv * 16, 16)] - soff
                for jj in range(16):
                    d_splat = _bcast(hdv, jj)
                    j = jv * 16 + jj
                    m0 = rows_buf[j, pl.ds(0, 16)]
                    m1 = rows_buf[j, pl.ds(16, 16)]
                    a0 = plsc.load_gather(mx_acc, [d_splat, lane])
                    plsc.store_scatter(mx_acc, [d_splat, lane],
                                       jnp.maximum(a0, m0))
                    a1 = plsc.load_gather(mx_acc, [d_splat, lane2])
                    plsc.store_scatter(mx_acc, [d_splat, lane2],
                                       jnp.maximum(a1, m1))
                    b0 = plsc.load_gather(mn_acc, [d_splat, lane])
                    plsc.store_scatter(mn_acc, [d_splat, lane],
                                       jnp.minimum(b0, m0))
                    b1 = plsc.load_gather(mn_acc, [d_splat, lane2])
                    plsc.store_scatter(mn_acc, [d_splat, lane2],
                                       jnp.minimum(b1, m1))
                    plsc.addupdate_scatter(cnt_acc, [d_splat], fones,
                                           mask=lane0)
                return 0
            lax.fori_loop(0, B // 16, edge_vreg, 0)

            sum_cp.wait()

            @plsc.parallel_loop(0, B, unroll=4)
            def sq_loop(j):
                m0 = rows_buf[j, pl.ds(0, 16)]
                m1 = rows_buf[j, pl.ds(16, 16)]
                rows_buf[j, pl.ds(0, 16)] = m0 * m0
                rows_buf[j, pl.ds(16, 16)] = m1 * m1

            pltpu.sync_copy(rows_buf, s2_sh.at[hdsh_buf], add=True)

        pltpu.async_copy(dstp.at[pl.ds(0, CH)], dst_buf.at[pl.ds(0, CH)],
                         sem2)

        def chunk_body(ci, ptr):
            par = pl.multiple_of((ci & 1) * CH, CH)
            cofs = pl.multiple_of(ci * CH, CH)
            pltpu.make_async_copy(dstp.at[pl.ds(cofs, CH)],
                                  dst_buf.at[pl.ds(par, CH)], sem2).wait()

            @pl.when(ci + 1 < NCHUNK)
            def _prefetch():
                nofs = pl.multiple_of((ci + 1) * CH, CH)
                npar = pl.multiple_of(((ci + 1) & 1) * CH, CH)
                pltpu.async_copy(dstp.at[pl.ds(nofs, CH)],
                                 dst_buf.at[pl.ds(npar, CH)], sem2)

            @plsc.parallel_loop(0, NV // 16, unroll=2)
            def l1(g):
                gofs = pl.multiple_of(par + g * 256, 256)
                acc = zi
                for k in range(16):
                    u = dst_buf[pl.ds(gofs + k * 16, 16)] - lo
                    m = (u >= 0) & (u < R)
                    pop = plsc.all_reduce_population_count(m)
                    acc = jnp.where(lane == k, pop, acc)
                pcnt_buf[pl.ds(g * 16, 16)] = acc

            def l2(g, carry):
                cg = pcnt_buf[pl.ds(g * 16, 16)]
                inc = plsc.cumsum(cg)
                pbase_buf[pl.ds(g * 16, 16)] = inc - cg + carry
                return carry + _bcast(inc, 15)
            tot = lax.fori_loop(0, NV // 16, l2, _splat(ptr))

            @plsc.parallel_loop(0, NV // 16, unroll=2)
            def l3(g):
                gofs = pl.multiple_of(par + g * 256, 256)
                pb = pbase_buf[pl.ds(g * 16, 16)]
                for k in range(16):
                    u = dst_buf[pl.ds(gofs + k * 16, 16)] - lo
                    m = (u >= 0) & (u < R)
                    inc = plsc.cumsum(jnp.where(m, 1, 0))
                    pos = _bcast(pb, k) + inc - 1
                    eid = _splat(cofs + g * 256 + k * 16) + lane
                    plsc.store_scatter(hs_buf, [pos], eid, mask=m)
                    plsc.store_scatter(hd_buf, [pos], u + soff, mask=m)

            ptr_new = jnp.sum(jnp.where(lane == 15, tot, 0))

            def drain_cond(st):
                return st[0] + B <= st[1]

            def drain_body(st):
                fofs, pn = st
                flush(fofs)
                return (fofs + B, pn)
            fofs, _ = lax.while_loop(drain_cond, drain_body,
                                     (jnp.int32(0), ptr_new))

            def compact(q):
                q = pl.multiple_of(q, B)
                def cp(j, _):
                    hs_buf[pl.ds(j * 16, 16)] = hs_buf[pl.ds(q + j * 16, 16)]
                    hd_buf[pl.ds(j * 16, 16)] = hd_buf[pl.ds(q + j * 16, 16)]
                    return 0
                lax.fori_loop(0, B // 16, cp, 0)
                return jnp.int32(0)
            fofs = lax.cond(fofs > 0, compact, lambda q: q, fofs)
            return ptr_new - (ptr_new // B) * B

        ptr = lax.fori_loop(0, NCHUNK, chunk_body, jnp.int32(0))

        def final_flush(q):
            def fix(j, _):
                gidx = lane + j * 16
                hd_buf[pl.ds(j * 16, 16)] = jnp.where(
                    gidx >= q, dump, hd_buf[pl.ds(j * 16, 16)])
                return 0
            lax.fori_loop(0, B // 16, fix, 0)
            flush(jnp.int32(0))
            return jnp.int32(0)
        ptr = lax.cond(ptr > 0, final_flush, lambda q: q, ptr)

        pltpu.sync_copy(mx_acc.at[pl.ds(0, R)], mx_out.at[pl.ds(lo, R)])
        pltpu.sync_copy(mn_acc.at[pl.ds(0, R)], mn_out.at[pl.ds(lo, R)])
        pltpu.sync_copy(cnt_acc.at[pl.ds(0, R)], cnt_out.at[pl.ds(lo, R)])
        pltpu.sync_copy(s_sh.at[pl.ds(soff, R)], s_out.at[pl.ds(lo, R)])
        pltpu.sync_copy(s2_sh.at[pl.ds(soff, R)], s2_out.at[pl.ds(lo, R)])
        return 0

    lax.fori_loop(0, NPASS, one_pass, 0)


_sc_call = pl.kernel(
    _sc_body,
    out_type=[
        jax.ShapeDtypeStruct((N_PAD, H), jnp.float32),
        jax.ShapeDtypeStruct((N_PAD, H), jnp.float32),
        jax.ShapeDtypeStruct((N_PAD, H), jnp.float32),
        jax.ShapeDtypeStruct((N_PAD, H), jnp.float32),
        jax.ShapeDtypeStruct((N_PAD,), jnp.float32),
    ],
    mesh=_mesh,
    compiler_params=pltpu.CompilerParams(
        use_tc_tiling_on_sc=False, needs_layout_passes=False),
    scratch_types=[
        pltpu.VMEM((2 * CH,), jnp.int32),
        pltpu.SemaphoreType.DMA,
        pltpu.VMEM((NV,), jnp.int32),
        pltpu.VMEM((NV,), jnp.int32),
        pltpu.VMEM((HB,), jnp.int32),
        pltpu.VMEM((HB,), jnp.int32),
        pltpu.VMEM((B,), jnp.int32),
        pltpu.VMEM((B,), jnp.int32),
        pltpu.VMEM((B, H), jnp.float32),
        pltpu.VMEM((R + 8, H), jnp.float32),
        pltpu.VMEM((R + 8, H), jnp.float32),
        pltpu.VMEM((800,), jnp.float32),
        pltpu.VMEM_SHARED((16 * RPAD, H), jnp.float32),
        pltpu.VMEM_SHARED((16 * RPAD, H), jnp.float32),
        pltpu.SemaphoreType.DMA,
    ],
)


def _tc_body(s_ref, s2_ref, mx_ref, mn_ref, cnt_ref, w_ref, b_ref, o_ref):
    cnt = cnt_ref[...]
    denom = jnp.maximum(cnt, 1.0)
    has = cnt > 0.0
    mean = s_ref[...] / denom
    mx = jnp.where(has, mx_ref[...], 0.0)
    mn = jnp.where(has, mn_ref[...], 0.0)
    msq = s2_ref[...] / denom
    var = jnp.maximum(msq - mean * mean, 0.0)
    std = jnp.sqrt(var + 1e-5)
    h = jnp.concatenate([mean, mx, mn, std], axis=1)
    o = jnp.dot(h, w_ref[...], preferred_element_type=jnp.float32) + b_ref[...]
    o_ref[...] = jnp.maximum(o, 0.0)


BN = 512
_tc_call = pl.pallas_call(
    _tc_body,
    grid=(N_PAD // BN,),
    in_specs=[
        pl.BlockSpec((BN, H), lambda i: (i, 0)),
        pl.BlockSpec((BN, H), lambda i: (i, 0)),
        pl.BlockSpec((BN, H), lambda i: (i, 0)),
        pl.BlockSpec((BN, H), lambda i: (i, 0)),
        pl.BlockSpec((BN, 1), lambda i: (i, 0)),
        pl.BlockSpec((4 * H, H), lambda i: (0, 0)),
        pl.BlockSpec((1, H), lambda i: (0, 0)),
    ],
    out_specs=pl.BlockSpec((BN, H), lambda i: (i, 0)),
    out_shape=jax.ShapeDtypeStruct((N_PAD, H), jnp.float32),
)


def kernel(feat, edge_index, W, b):
    src = edge_index[0].astype(jnp.int32)
    dst = edge_index[1].astype(jnp.int32)
    pad = E_PAD - E
    srcp = jnp.concatenate([src, jnp.zeros((pad,), jnp.int32)])
    dstp = jnp.concatenate([dst, jnp.full((pad,), N_PAD - 1, jnp.int32)])
    s_, s2_, mx_, mn_, cnt_ = _sc_call(feat, srcp, dstp)
    out = _tc_call(s_, s2_, mx_, mn_, cnt_.reshape(N_PAD, 1), W,
                   b.reshape(1, H))
    return out[:N_NODES]
